# Initial kernel scaffold; baseline (speedup 1.0000x reference)
import jax
import jax.numpy as jnp
from jax.experimental import pallas as pl


def kernel(x, edge_index, edge_attr, emb_table, We0, be0, W1_0, b1_0, gamma0, beta0, W2_0, b2_0, We1, be1, W1_1, b1_1, gamma1, beta1, W2_1, b2_1):
    raise NotImplementedError("write your pallas kernel here")


# speedup vs baseline: 4.2408x; 4.2408x over previous
import functools

import jax
import jax.numpy as jnp
from jax import lax
from jax.experimental import pallas as pl
from jax.experimental.pallas import tpu as pltpu
from jax.experimental.pallas import tpu_sc as plsc

N = 10000
E = 160000
D = 128
NPAD = 10112
STRIPE = NPAD // 16
EPAD = 163840
EPT = EPAD // 32
CHUNK = 128
NCHUNK = EPT // CHUNK

_f32 = jnp.float32
_mesh = plsc.VectorSubcoreMesh(core_axis_name="c", subcore_axis_name="s")


@functools.partial(
    pl.kernel,
    out_type=jax.ShapeDtypeStruct((2, NPAD, 128), _f32),
    mesh=_mesh,
    scratch_types=[
        pltpu.VMEM((CHUNK,), jnp.int32),
        pltpu.VMEM((CHUNK,), jnp.int32),
        pltpu.VMEM((CHUNK, 128), _f32),
        pltpu.VMEM((NPAD,), _f32),
        pltpu.VMEM_SHARED((NPAD, 128), _f32),
        pltpu.SemaphoreType.DMA,
    ],
    compiler_params=pltpu.CompilerParams(needs_layout_passes=False),
)
def _sc_edge_stats(colp, rowp, ea128, xf_h, z128, outAB,
                   idxv, ridxv, buf, xfv, acc, sem):
    c = lax.axis_index("c")
    s = lax.axis_index("s")
    wid = s * 2 + c
    stripe = pl.ds(s * STRIPE, STRIPE)
    pltpu.sync_copy(xf_h, xfv)
    pltpu.sync_copy(z128.at[stripe], acc.at[stripe])
    plsc.subcore_barrier()

    lanes = lax.iota(jnp.int32, 16)
    col17 = jnp.full((16,), 17, jnp.int32)

    def body(i, carry):
        base = wid * EPT + i * CHUNK
        pltpu.sync_copy(colp.at[pl.ds(base, CHUNK)], idxv)
        pltpu.sync_copy(rowp.at[pl.ds(base, CHUNK)], ridxv)
        pltpu.sync_copy(ea128.at[pl.ds(base, CHUNK)], buf)
        for g in range(CHUNK // 16):
            cols16 = idxv[pl.ds(g * 16, 16)]
            xs = plsc.load_gather(xfv, [cols16])
            plsc.store_scatter(buf, [lanes + g * 16, col17], xs)
        pltpu.sync_copy(buf, acc.at[ridxv], add=True)
        return carry

    lax.fori_loop(0, NCHUNK, body, 0)
    plsc.subcore_barrier()
    pltpu.sync_copy(acc.at[stripe], outAB.at[c, stripe])


@functools.partial(
    pl.kernel,
    out_type=jax.ShapeDtypeStruct((2, NPAD, D), _f32),
    mesh=_mesh,
    scratch_types=[
        pltpu.VMEM((CHUNK,), jnp.int32),
        pltpu.VMEM((CHUNK,), jnp.int32),
        pltpu.VMEM((CHUNK, D), _f32),
        pltpu.VMEM_SHARED((NPAD, D), _f32),
        pltpu.SemaphoreType.DMA,
    ],
)
def _sc_gather_sum(colp, rowp, h1p, z128, outG, idxv, ridxv, rows, acc, sem):
    c = lax.axis_index("c")
    s = lax.axis_index("s")
    wid = s * 2 + c
    stripe = pl.ds(s * STRIPE, STRIPE)

    @pl.when(c == 0)
    def _():
        pltpu.sync_copy(h1p.at[stripe], acc.at[stripe])

    @pl.when(c != 0)
    def _():
        pltpu.sync_copy(z128.at[stripe], acc.at[stripe])

    plsc.subcore_barrier()

    def body(i, carry):
        base = wid * EPT + i * CHUNK
        pltpu.sync_copy(colp.at[pl.ds(base, CHUNK)], idxv)
        pltpu.sync_copy(rowp.at[pl.ds(base, CHUNK)], ridxv)
        pltpu.async_copy(h1p.at[idxv], rows, sem).wait()
        pltpu.sync_copy(rows, acc.at[ridxv], add=True)
        return carry

    lax.fori_loop(0, NCHUNK, body, 0)
    plsc.subcore_barrier()
    pltpu.sync_copy(acc.at[stripe], outG.at[c, stripe])


def _bn_relu_proj(z, gamma, beta, W2, b2):
    zv = z[:N]
    m = jnp.mean(zv, axis=0, keepdims=True)
    v = jnp.mean((zv - m) * (zv - m), axis=0, keepdims=True)
    zn = (z - m) / jnp.sqrt(v + 1e-5) * gamma + beta
    zn = jnp.maximum(zn, 0.0)
    return jnp.dot(zn, W2, preferred_element_type=_f32, precision=lax.Precision.HIGHEST) + b2


def _tc1_body(outAB, xfp, emb, We0p, be0, W1, b1, gamma, beta, W2, b2,
              h1_out):
    AB = outAB[0] + outAB[1]
    A = AB[:, 0:16]
    dd = AB[:, 16:17] + 1.0
    q = AB[:, 17:18] + xfp[...]
    W1m = W1[...]
    W1a = W1m[:D]
    W1b = W1m[D:]
    emb0 = emb[0:1, :]
    emb1 = emb[1:2, :]
    M16 = jnp.dot(We0p[...], W1b, preferred_element_type=_f32, precision=lax.Precision.HIGHEST)
    r0 = jnp.dot(We0p[7:8, :], W1b, preferred_element_type=_f32, precision=lax.Precision.HIGHEST) + b1[...]
    r1 = (jnp.dot(emb0, W1a, preferred_element_type=_f32, precision=lax.Precision.HIGHEST)
          + jnp.dot(be0[...], W1b, preferred_element_type=_f32, precision=lax.Precision.HIGHEST))
    r2 = jnp.dot(emb1 - emb0, W1a, preferred_element_type=_f32, precision=lax.Precision.HIGHEST)
    z = (jnp.dot(A, M16, preferred_element_type=_f32)
         + r0 + dd * r1 + q * r2)
    h1 = _bn_relu_proj(z, gamma[...], beta[...], W2[...], b2[...])
    h1_out[...] = jnp.maximum(h1, 0.0)


def _tc2_body(outG, outAB, We1p, be1, W1, b1, gamma, beta, W2, b2,
              out):
    G = outG[0] + outG[1]
    AB = outAB[0] + outAB[1]
    A = AB[:, 0:16]
    dd = AB[:, 16:17] + 1.0
    W1m = W1[...]
    W1a = W1m[:D]
    W1b = W1m[D:]
    M16 = jnp.dot(We1p[...], W1b, preferred_element_type=_f32, precision=lax.Precision.HIGHEST)
    r0 = jnp.dot(We1p[7:8, :], W1b, preferred_element_type=_f32, precision=lax.Precision.HIGHEST) + b1[...]
    rd = jnp.dot(be1[...], W1b, preferred_element_type=_f32, precision=lax.Precision.HIGHEST)
    z = (jnp.dot(G, W1a, preferred_element_type=_f32, precision=lax.Precision.HIGHEST)
         + jnp.dot(A, M16, preferred_element_type=_f32)
         + r0 + dd * rd)
    out[...] = _bn_relu_proj(z, gamma[...], beta[...], W2[...], b2[...])


_tc1 = pl.pallas_call(
    _tc1_body, out_shape=jax.ShapeDtypeStruct((NPAD, D), _f32))
_tc2 = pl.pallas_call(
    _tc2_body, out_shape=jax.ShapeDtypeStruct((NPAD, D), _f32))


def kernel(x, edge_index, edge_attr, emb_table,
           We0, be0, W1_0, b1_0, gamma0, beta0, W2_0, b2_0,
           We1, be1, W1_1, b1_1, gamma1, beta1, W2_1, b2_1):
    xf = x.astype(_f32)
    col = edge_index[1].astype(jnp.int32)
    row = edge_index[0].astype(jnp.int32)
    colp = jnp.concatenate([col, jnp.zeros((EPAD - E,), jnp.int32)])
    rowp = jnp.concatenate([row, jnp.full((EPAD - E,), N + 8, jnp.int32)])
    ea128 = jnp.zeros((EPAD, 128), _f32)
    ea128 = ea128.at[:E, 0:9].set(edge_attr.astype(_f32))
    ea128 = ea128.at[:, 16].set(1.0)
    xf_h = jnp.concatenate([xf, jnp.zeros((NPAD - N,), _f32)])
    z128 = jnp.zeros((NPAD, D), _f32)
    xfp = jnp.concatenate([xf[:, None], jnp.zeros((NPAD - N, 1), _f32)],
                          axis=0)

    outAB = _sc_edge_stats(colp, rowp, ea128, xf_h, z128)
    outAB = outAB[:, :, 0:24]

    We0p = jnp.concatenate([We0.astype(_f32), jnp.zeros((7, D), _f32)], axis=0)
    We1p = jnp.concatenate([We1.astype(_f32), jnp.zeros((7, D), _f32)], axis=0)
    h1p = _tc1(outAB, xfp, emb_table, We0p, be0[None, :], W1_0,
               b1_0[None, :], gamma0[None, :], beta0[None, :], W2_0,
               b2_0[None, :])

    outG = _sc_gather_sum(colp, rowp, h1p, z128)

    outf = _tc2(outG, outAB, We1p, be1[None, :], W1_1,
                b1_1[None, :], gamma1[None, :], beta1[None, :], W2_1,
                b2_1[None, :])
    return outf[:N]


# speedup vs baseline: 4.5381x; 1.0701x over previous
import functools

import jax
import jax.numpy as jnp
from jax import lax
from jax.experimental import pallas as pl
from jax.experimental.pallas import tpu as pltpu
from jax.experimental.pallas import tpu_sc as plsc

N = 10000
E = 160000
D = 128
NPAD = 10112
STRIPE = NPAD // 16
EPAD = 163840
EPT = EPAD // 32
CHUNK = 128
NCHUNK = EPT // CHUNK

_f32 = jnp.float32
_mesh = plsc.VectorSubcoreMesh(core_axis_name="c", subcore_axis_name="s")


@functools.partial(
    pl.kernel,
    out_type=jax.ShapeDtypeStruct((2, NPAD, 128), _f32),
    mesh=_mesh,
    scratch_types=[
        pltpu.VMEM((CHUNK,), jnp.int32),
        pltpu.VMEM((CHUNK,), jnp.int32),
        pltpu.VMEM((CHUNK, 128), _f32),
        pltpu.VMEM((NPAD,), _f32),
        pltpu.VMEM_SHARED((NPAD, 128), _f32),
        pltpu.SemaphoreType.DMA,
    ],
    compiler_params=pltpu.CompilerParams(needs_layout_passes=False),
)
def _sc_edge_stats(colp, rowp, ea128, xf_h, z128, outAB,
                   idxv, ridxv, buf, xfv, acc, sem):
    c = lax.axis_index("c")
    s = lax.axis_index("s")
    wid = s * 2 + c
    stripe = pl.ds(s * STRIPE, STRIPE)
    pltpu.sync_copy(xf_h, xfv)
    pltpu.sync_copy(z128.at[stripe], acc.at[stripe])
    plsc.subcore_barrier()

    lanes = lax.iota(jnp.int32, 16)
    col17 = jnp.full((16,), 17, jnp.int32)

    def body(i, carry):
        base = wid * EPT + i * CHUNK
        pltpu.sync_copy(colp.at[pl.ds(base, CHUNK)], idxv)
        pltpu.sync_copy(rowp.at[pl.ds(base, CHUNK)], ridxv)
        pltpu.sync_copy(ea128.at[pl.ds(base, CHUNK)], buf)
        for g in range(CHUNK // 16):
            cols16 = idxv[pl.ds(g * 16, 16)]
            xs = plsc.load_gather(xfv, [cols16])
            plsc.store_scatter(buf, [lanes + g * 16, col17], xs)
        pltpu.sync_copy(buf, acc.at[ridxv], add=True)
        return carry

    lax.fori_loop(0, NCHUNK, body, 0)
    plsc.subcore_barrier()
    pltpu.sync_copy(acc.at[stripe], outAB.at[c, stripe])


_NBUF = 4


@functools.partial(
    pl.kernel,
    out_type=jax.ShapeDtypeStruct((2, NPAD, D), _f32),
    mesh=_mesh,
    scratch_types=[
        [pltpu.VMEM((CHUNK,), jnp.int32) for _ in range(_NBUF)],
        [pltpu.VMEM((CHUNK,), jnp.int32) for _ in range(_NBUF)],
        [pltpu.VMEM((CHUNK, D), _f32) for _ in range(2)],
        [pltpu.SemaphoreType.DMA for _ in range(_NBUF)],
        [pltpu.SemaphoreType.DMA for _ in range(2)],
        pltpu.VMEM_SHARED((NPAD, D), _f32),
    ],
)
def _sc_gather_sum(colp, rowp, h1p, z128, outG,
                   cidx, ridx, rows, isem, gsem, acc):
    c = lax.axis_index("c")
    s = lax.axis_index("s")
    wid = s * 2 + c
    stripe = pl.ds(s * STRIPE, STRIPE)
    pltpu.sync_copy(z128.at[stripe], acc.at[stripe])
    plsc.subcore_barrier()

    gd = [None] * NCHUNK

    def stage_i(i):
        b = i % _NBUF
        base = wid * EPT + i * CHUNK
        d1 = pltpu.async_copy(colp.at[pl.ds(base, CHUNK)], cidx[b], isem[b])
        d2 = pltpu.async_copy(rowp.at[pl.ds(base, CHUNK)], ridx[b], isem[b])
        d1.wait()
        d2.wait()

    def stage_g(i):
        gd[i] = pltpu.async_copy(h1p.at[cidx[i % _NBUF]], rows[i % 2],
                                 gsem[i % 2])

    def stage_s(i):
        gd[i].wait()
        pltpu.sync_copy(rows[i % 2], acc.at[ridx[i % _NBUF]], add=True)

    for i in range(NCHUNK + 1):
        if i < NCHUNK:
            stage_i(i)
            stage_g(i)
        if 0 <= i - 1 < NCHUNK:
            stage_s(i - 1)

    plsc.subcore_barrier()
    pltpu.sync_copy(acc.at[stripe], outG.at[c, stripe])


def _bn_relu_proj(z, gamma, beta, W2, b2):
    zv = z[:N]
    m = jnp.mean(zv, axis=0, keepdims=True)
    v = jnp.mean((zv - m) * (zv - m), axis=0, keepdims=True)
    zn = (z - m) / jnp.sqrt(v + 1e-5) * gamma + beta
    zn = jnp.maximum(zn, 0.0)
    return jnp.dot(zn, W2, preferred_element_type=_f32,
                   precision=lax.Precision.HIGHEST) + b2


def _tc1_body(outAB, xfp, emb, We0p, be0, W1, b1, gamma, beta, W2, b2,
              h1_out):
    AB = outAB[0] + outAB[1]
    A = AB[:, 0:16]
    dd = AB[:, 16:17] + 1.0
    q = AB[:, 17:18] + xfp[...]
    W1m = W1[...]
    W1a = W1m[:D]
    W1b = W1m[D:]
    emb0 = emb[0:1, :]
    emb1 = emb[1:2, :]
    M16 = jnp.dot(We0p[...], W1b, preferred_element_type=_f32,
                  precision=lax.Precision.HIGHEST)
    r0 = jnp.dot(We0p[7:8, :], W1b, preferred_element_type=_f32,
                 precision=lax.Precision.HIGHEST) + b1[...]
    r1 = (jnp.dot(emb0, W1a, preferred_element_type=_f32,
                  precision=lax.Precision.HIGHEST)
          + jnp.dot(be0[...], W1b, preferred_element_type=_f32,
                    precision=lax.Precision.HIGHEST))
    r2 = jnp.dot(emb1 - emb0, W1a, preferred_element_type=_f32,
                 precision=lax.Precision.HIGHEST)
    z = (jnp.dot(A, M16, preferred_element_type=_f32)
         + r0 + dd * r1 + q * r2)
    h1 = _bn_relu_proj(z, gamma[...], beta[...], W2[...], b2[...])
    h1_out[...] = jnp.maximum(h1, 0.0)


def _tc2_body(outG, h1p, outAB, We1p, be1, W1, b1, gamma, beta, W2, b2,
              out):
    G = outG[0] + outG[1] + h1p[...]
    AB = outAB[0] + outAB[1]
    A = AB[:, 0:16]
    dd = AB[:, 16:17] + 1.0
    W1m = W1[...]
    W1a = W1m[:D]
    W1b = W1m[D:]
    M16 = jnp.dot(We1p[...], W1b, preferred_element_type=_f32,
                  precision=lax.Precision.HIGHEST)
    r0 = jnp.dot(We1p[7:8, :], W1b, preferred_element_type=_f32,
                 precision=lax.Precision.HIGHEST) + b1[...]
    rd = jnp.dot(be1[...], W1b, preferred_element_type=_f32,
                 precision=lax.Precision.HIGHEST)
    z = (jnp.dot(G, W1a, preferred_element_type=_f32,
                 precision=lax.Precision.HIGHEST)
         + jnp.dot(A, M16, preferred_element_type=_f32)
         + r0 + dd * rd)
    out[...] = _bn_relu_proj(z, gamma[...], beta[...], W2[...], b2[...])


_tc1 = pl.pallas_call(
    _tc1_body, out_shape=jax.ShapeDtypeStruct((NPAD, D), _f32))
_tc2 = pl.pallas_call(
    _tc2_body, out_shape=jax.ShapeDtypeStruct((NPAD, D), _f32))


def kernel(x, edge_index, edge_attr, emb_table,
           We0, be0, W1_0, b1_0, gamma0, beta0, W2_0, b2_0,
           We1, be1, W1_1, b1_1, gamma1, beta1, W2_1, b2_1):
    xf = x.astype(_f32)
    col = edge_index[1].astype(jnp.int32)
    row = edge_index[0].astype(jnp.int32)
    colp = jnp.concatenate([col, jnp.zeros((EPAD - E,), jnp.int32)])
    rowp = jnp.concatenate([row, jnp.full((EPAD - E,), N + 8, jnp.int32)])
    ea128 = jnp.zeros((EPAD, 128), _f32)
    ea128 = ea128.at[:E, 0:9].set(edge_attr.astype(_f32))
    ea128 = ea128.at[:, 16].set(1.0)
    xf_h = jnp.concatenate([xf, jnp.zeros((NPAD - N,), _f32)])
    z128 = jnp.zeros((NPAD, D), _f32)
    xfp = jnp.concatenate([xf[:, None], jnp.zeros((NPAD - N, 1), _f32)],
                          axis=0)

    outAB = _sc_edge_stats(colp, rowp, ea128, xf_h, z128)
    outAB = outAB[:, :, 0:24]

    We0p = jnp.concatenate([We0.astype(_f32), jnp.zeros((7, D), _f32)], axis=0)
    We1p = jnp.concatenate([We1.astype(_f32), jnp.zeros((7, D), _f32)], axis=0)
    h1p = _tc1(outAB, xfp, emb_table, We0p, be0[None, :], W1_0,
               b1_0[None, :], gamma0[None, :], beta0[None, :], W2_0,
               b2_0[None, :])

    outG = _sc_gather_sum(colp, rowp, h1p, z128)

    outf = _tc2(outG, h1p, outAB, We1p, be1[None, :], W1_1,
                b1_1[None, :], gamma1[None, :], beta1[None, :], W2_1,
                b2_1[None, :])
    return outf[:N]
